# Initial kernel scaffold; baseline (speedup 1.0000x reference)
import jax
import jax.numpy as jnp
from jax.experimental import pallas as pl


def kernel(node_features, lattices, edge_index, edge2graph, frac_diff, edge_feats, ln_scale, ln_bias, edge_W1, edge_b1, edge_W2, edge_b2, node_W1, node_b1, node_W2, node_b2):
    raise NotImplementedError("write your pallas kernel here")


# speedup vs baseline: 3.1322x; 3.1322x over previous
import functools
import math

import jax
import jax.numpy as jnp
from jax import lax
from jax.experimental import pallas as pl
from jax.experimental.pallas import tpu as pltpu
from jax.experimental.pallas import tpu_sc as plsc

F32 = jnp.float32
H = 128
N_NODES = 10000
N_EDGES = 160000
N_GRAPHS = 500
N_FREQ = 64

NC, NS = 2, 16
NW = NC * NS
CH = 128
EPW = 5120
E_PAD = NW * EPW
NCHUNK = EPW // CH
G_PAD = 512
GT_W = 144
NR = 10240
RPT = NR // NS
DUMMY = N_NODES + 8

E_BLK = 2048
EGRID = E_PAD // E_BLK
NB = 1024
NGRID = NR // NB
NODE_BLK = 1000
PGRID = N_NODES // NODE_BLK


def _prep_nodes_body(nf_ref, lns_ref, lnb_ref, whi_ref, whj_ref,
                     h_ref, pi_ref, pj_ref):
    x = nf_ref[...]
    mu = jnp.mean(x, axis=1, keepdims=True)
    xc = x - mu
    var = jnp.mean(xc * xc, axis=1, keepdims=True)
    h = xc * lax.rsqrt(var + 1e-5) * lns_ref[...] + lnb_ref[...]
    h_ref[...] = h
    pi_ref[...] = jnp.dot(h, whi_ref[...], preferred_element_type=F32)
    pj_ref[...] = jnp.dot(h, whj_ref[...], preferred_element_type=F32)


_prep_nodes = pl.pallas_call(
    _prep_nodes_body,
    grid=(PGRID,),
    in_specs=[
        pl.BlockSpec((NODE_BLK, H), lambda i: (i, 0)),
        pl.BlockSpec((1, H), lambda i: (0, 0)),
        pl.BlockSpec((1, H), lambda i: (0, 0)),
        pl.BlockSpec((H, H), lambda i: (0, 0)),
        pl.BlockSpec((H, H), lambda i: (0, 0)),
    ],
    out_specs=[pl.BlockSpec((NODE_BLK, H), lambda i: (i, 0))] * 3,
    out_shape=[jax.ShapeDtypeStruct((N_NODES, H), F32)] * 3,
)


def _graph_body(lat_ref, wlat_ref, b1_ref, gt_ref):
    lat = lat_ref[...]
    ar = lat[:, 3:6] * (math.pi / 180.0)
    c = jnp.cos(ar)
    s = jnp.sin(ar)
    c0, c1, c2 = c[:, 0:1], c[:, 1:2], c[:, 2:3]
    s0, s1 = s[:, 0:1], s[:, 1:2]
    val = jnp.clip((c0 * c1 - c2) / (s0 * s1), -1.0, 1.0)
    sg = jnp.sqrt(jnp.maximum(1.0 - val * val, 0.0))
    zero = jnp.zeros_like(val)
    one = jnp.ones_like(val)
    mp = jnp.concatenate(
        [s1, zero, c1,
         -s0 * val, s0 * sg, c0,
         zero, zero, one], axis=1)
    glat = jnp.dot(lat, wlat_ref[...], preferred_element_type=F32) + b1_ref[...]
    gt_ref[...] = jnp.concatenate(
        [glat, mp, jnp.zeros((G_PAD, GT_W - H - 9), F32)], axis=1)


_graph_table = pl.pallas_call(
    _graph_body,
    out_shape=jax.ShapeDtypeStruct((G_PAD, GT_W), F32),
)


def _edge_body(ef_ref, pi_ref, pj_ref, fd_ref, e2g_ref, gt_ref,
               w1e_ref, w1s_ref, w1c_ref, w2_ref, b2_ref, out_ref):
    e2g = e2g_ref[...]
    iota_g = lax.broadcasted_iota(jnp.int32, (E_BLK, G_PAD), 1)
    oh = jnp.where(e2g == iota_g, 1.0, 0.0)
    g = jnp.dot(oh, gt_ref[...], preferred_element_type=F32)
    z = jnp.dot(ef_ref[...], w1e_ref[...], preferred_element_type=F32)
    z = z + pi_ref[...] + pj_ref[...] + g[:, 0:H]
    fd = fd_ref[...]
    fr = 2.0 * math.pi * lax.broadcasted_iota(
        jnp.int32, (1, N_FREQ), 1).astype(F32)
    xs = []
    for k in range(3):
        xk = (fd[:, 0:1] * g[:, H + 3 * k:H + 3 * k + 1]
              + fd[:, 1:2] * g[:, H + 3 * k + 1:H + 3 * k + 2]
              + fd[:, 2:3] * g[:, H + 3 * k + 2:H + 3 * k + 3])
        xs.append(xk * fr)
    X = jnp.concatenate(xs, axis=1)
    z = z + jnp.dot(jnp.sin(X), w1s_ref[...], preferred_element_type=F32)
    z = z + jnp.dot(jnp.cos(X), w1c_ref[...], preferred_element_type=F32)
    m = z * jax.nn.sigmoid(z)
    m = jnp.dot(m, w2_ref[...], preferred_element_type=F32) + b2_ref[...]
    m = m * jax.nn.sigmoid(m)
    out_ref[...] = jnp.concatenate(
        [m, jnp.ones((E_BLK, 1), F32), jnp.zeros((E_BLK, GT_W - H - 1), F32)],
        axis=1)


_edge_mlp = pl.pallas_call(
    _edge_body,
    grid=(EGRID,),
    in_specs=[
        pl.BlockSpec((E_BLK, H), lambda i: (i, 0)),
        pl.BlockSpec((E_BLK, H), lambda i: (i, 0)),
        pl.BlockSpec((E_BLK, H), lambda i: (i, 0)),
        pl.BlockSpec((E_BLK, 4), lambda i: (i, 0)),
        pl.BlockSpec((E_BLK, 1), lambda i: (i, 0)),
        pl.BlockSpec((G_PAD, GT_W), lambda i: (0, 0)),
        pl.BlockSpec((H, H), lambda i: (0, 0)),
        pl.BlockSpec((192, H), lambda i: (0, 0)),
        pl.BlockSpec((192, H), lambda i: (0, 0)),
        pl.BlockSpec((H, H), lambda i: (0, 0)),
        pl.BlockSpec((1, H), lambda i: (0, 0)),
    ],
    out_specs=pl.BlockSpec((E_BLK, GT_W), lambda i: (i, 0)),
    out_shape=jax.ShapeDtypeStruct((E_PAD, GT_W), F32),
)


def _node_body(h_ref, nf_ref, agg_ref, wna_ref, wnb_ref, nb1_ref,
               wn2_ref, nb2_ref, out_ref):
    a0 = agg_ref[0]
    a1 = agg_ref[1]
    ssum = a0[:, 0:H] + a1[:, 0:H]
    cnt = a0[:, H:H + 1] + a1[:, H:H + 1]
    agg = ssum / jnp.maximum(cnt, 1.0)
    t = (jnp.dot(h_ref[...], wna_ref[...], preferred_element_type=F32)
         + jnp.dot(agg, wnb_ref[...], preferred_element_type=F32)
         + nb1_ref[...])
    t = t * jax.nn.sigmoid(t)
    o = jnp.dot(t, wn2_ref[...], preferred_element_type=F32) + nb2_ref[...]
    o = o * jax.nn.sigmoid(o)
    out_ref[...] = nf_ref[...] + o


_node_mlp = pl.pallas_call(
    _node_body,
    grid=(NGRID,),
    in_specs=[
        pl.BlockSpec((NB, H), lambda i: (i, 0)),
        pl.BlockSpec((NB, H), lambda i: (i, 0)),
        pl.BlockSpec((2, NB, GT_W), lambda i: (0, i, 0)),
        pl.BlockSpec((H, H), lambda i: (0, 0)),
        pl.BlockSpec((H, H), lambda i: (0, 0)),
        pl.BlockSpec((1, H), lambda i: (0, 0)),
        pl.BlockSpec((H, H), lambda i: (0, 0)),
        pl.BlockSpec((1, H), lambda i: (0, 0)),
    ],
    out_specs=pl.BlockSpec((NB, H), lambda i: (i, 0)),
    out_shape=jax.ShapeDtypeStruct((NR, H), F32),
)


def _sc_gather_body(pi_hbm, pj_hbm, src_hbm, dst_hbm, oi_hbm, oj_hbm,
                    si, di, bi, bj, s1, s2):
    wid = lax.axis_index("s") * NC + lax.axis_index("c")
    base = wid * EPW

    def body(j, carry):
        off = base + j * CH
        pltpu.sync_copy(src_hbm.at[pl.ds(off, CH)], si)
        pltpu.sync_copy(dst_hbm.at[pl.ds(off, CH)], di)
        ci = pltpu.async_copy(pi_hbm.at[si], bi, s1)
        cj = pltpu.async_copy(pj_hbm.at[di], bj, s2)
        ci.wait()
        cj.wait()
        pltpu.sync_copy(bi, oi_hbm.at[pl.ds(off, CH)])
        pltpu.sync_copy(bj, oj_hbm.at[pl.ds(off, CH)])
        return carry

    lax.fori_loop(0, NCHUNK, body, 0)


@functools.lru_cache(maxsize=None)
def _sc_gather_call():
    mesh = plsc.VectorSubcoreMesh(
        core_axis_name="c", subcore_axis_name="s",
        num_cores=NC, num_subcores=NS)
    return pl.kernel(
        _sc_gather_body,
        out_type=(jax.ShapeDtypeStruct((E_PAD, H), F32),
                  jax.ShapeDtypeStruct((E_PAD, H), F32)),
        mesh=mesh,
        scratch_types=[
            pltpu.VMEM((CH,), jnp.int32),
            pltpu.VMEM((CH,), jnp.int32),
            pltpu.VMEM((CH, H), F32),
            pltpu.VMEM((CH, H), F32),
            pltpu.SemaphoreType.DMA,
            pltpu.SemaphoreType.DMA,
        ],
    )


def _sc_gather(pi, pj, src_g, dst_g):
    return _sc_gather_call()(pi, pj, src_g, dst_g)


def _sc_scatter_body(m_hbm, srcs_hbm, z_hbm, out_hbm, idxv, mbuf, shared):
    cid = lax.axis_index("c")
    sid = lax.axis_index("s")
    wid = sid * NC + cid
    pltpu.sync_copy(z_hbm, shared.at[pl.ds(sid * RPT, RPT)])
    pltpu.sync_copy(srcs_hbm.at[pl.ds(wid * NCHUNK, NCHUNK)], idxv)
    plsc.subcore_barrier()

    def body(j, carry):
        pltpu.sync_copy(m_hbm.at[pl.ds(wid * EPW + j * CH, CH)], mbuf)
        pltpu.sync_copy(mbuf, shared.at[idxv.at[j]], add=True)
        return carry

    lax.fori_loop(0, NCHUNK, body, 0)
    plsc.subcore_barrier()
    pltpu.sync_copy(shared.at[pl.ds(sid * RPT, RPT)],
                    out_hbm.at[pl.ds(cid * NR + sid * RPT, RPT)])


@functools.lru_cache(maxsize=None)
def _sc_scatter_call():
    mesh = plsc.VectorSubcoreMesh(
        core_axis_name="c", subcore_axis_name="s",
        num_cores=NC, num_subcores=NS)
    return pl.kernel(
        _sc_scatter_body,
        out_type=jax.ShapeDtypeStruct((NC * NR, GT_W), F32),
        mesh=mesh,
        compiler_params=pltpu.CompilerParams(use_tc_tiling_on_sc=False),
        scratch_types=[
            pltpu.VMEM((NCHUNK, CH), jnp.int32),
            pltpu.VMEM((CH, GT_W), F32),
            pltpu.VMEM_SHARED((NR, GT_W), F32),
        ],
    )


def _sc_scatter(m_ext, src_s, zrows):
    return _sc_scatter_call()(m_ext, src_s, zrows)


def kernel(node_features, lattices, edge_index, edge2graph, frac_diff,
           edge_feats, ln_scale, ln_bias, edge_W1, edge_b1, edge_W2, edge_b2,
           node_W1, node_b1, node_W2, node_b2):
    w1e = edge_W1[0:H]
    whi = edge_W1[H:2 * H]
    whj = edge_W1[2 * H:3 * H]
    wlat = jnp.concatenate([edge_W1[3 * H:3 * H + 6], jnp.zeros((2, H), F32)], 0)
    wsin = edge_W1[390:582]
    wcos = edge_W1[582:774]

    lat_pad = jnp.concatenate(
        [lattices,
         jnp.tile(jnp.asarray([[1.0, 1.0, 1.0, 90.0, 90.0, 90.0]], F32),
                  (G_PAD - N_GRAPHS, 1))], 0)
    lat8 = jnp.concatenate([lat_pad, jnp.zeros((G_PAD, 2), F32)], 1)

    pad_e = E_PAD - N_EDGES
    src = edge_index[0]
    dst = edge_index[1]
    src_g = jnp.concatenate([src, jnp.zeros((pad_e,), jnp.int32)])
    dst_g = jnp.concatenate([dst, jnp.zeros((pad_e,), jnp.int32)])
    src_s = jnp.concatenate(
        [src, jnp.full((pad_e,), DUMMY, jnp.int32)]).reshape(E_PAD // CH, CH)
    e2g2d = jnp.concatenate(
        [edge2graph, jnp.zeros((pad_e,), jnp.int32)])[:, None]
    fd4 = jnp.concatenate([frac_diff, jnp.zeros((N_EDGES, 1), F32)], 1)
    fd4 = jnp.concatenate([fd4, jnp.zeros((pad_e, 4), F32)], 0)
    ef_pad = jnp.concatenate([edge_feats, jnp.zeros((pad_e, H), F32)], 0)

    h, pi, pj = _prep_nodes(node_features, ln_scale[None], ln_bias[None],
                            whi, whj)
    gtable = _graph_table(lat8, wlat, edge_b1[None])

    pig, pjg = _sc_gather(pi, pj, src_g, dst_g)

    m_ext = _edge_mlp(ef_pad, pig, pjg, fd4, e2g2d, gtable,
                      w1e, wsin, wcos, edge_W2, edge_b2[None])

    zrows = jnp.zeros((RPT, GT_W), F32)
    agg2 = _sc_scatter(m_ext, src_s, zrows).reshape(NC, NR, GT_W)

    pad_n = NR - N_NODES
    h_pad = jnp.concatenate([h, jnp.zeros((pad_n, H), F32)], 0)
    nf_pad = jnp.concatenate([node_features, jnp.zeros((pad_n, H), F32)], 0)
    out = _node_mlp(h_pad, nf_pad, agg2, node_W1[:H], node_W1[H:],
                    node_b1[None], node_W2, node_b2[None])
    return out[:N_NODES]
